# Initial kernel scaffold; baseline (speedup 1.0000x reference)
import jax
import jax.numpy as jnp
from jax.experimental import pallas as pl


def kernel(x, edge_index, weight):
    raise NotImplementedError("write your pallas kernel here")


# speedup vs baseline: 67.6717x; 67.6717x over previous
import functools

import jax
import jax.numpy as jnp
from jax import lax
from jax.experimental import pallas as pl
from jax.experimental.pallas import tpu as pltpu
from jax.experimental.pallas import tpu_sc as plsc

_K_FRAC_NUM, _K_FRAC_DEN = 1, 2


def _edge_remap(N, E, interpret=False):
    NC, NS, L = 2, 16, 16
    NW = NC * NS
    EW = E // NW
    W = 2000 if EW % 2000 == 0 else EW
    NWIN = EW // W
    mesh = plsc.VectorSubcoreMesh(
        core_axis_name="c", subcore_axis_name="s", num_cores=NC, num_subcores=NS
    )

    @functools.partial(
        pl.kernel,
        out_type=jax.ShapeDtypeStruct((2 * E,), jnp.int32),
        mesh=mesh,
        scratch_types=[
            pltpu.VMEM((N,), jnp.int32),
            pltpu.VMEM((W,), jnp.int32),
            pltpu.VMEM((W,), jnp.int32),
            pltpu.VMEM((W,), jnp.int32),
            pltpu.VMEM((W,), jnp.int32),
        ],
        compiler_params=pltpu.CompilerParams(needs_layout_passes=False),
        interpret=interpret,
    )
    def k(nm_hbm, ei_hbm, out_hbm, nm_v, src_v, dst_v, outr_v, outc_v):
        wid = lax.axis_index("s") * NC + lax.axis_index("c")
        pltpu.sync_copy(nm_hbm, nm_v)

        @pl.loop(0, NWIN)
        def _win(w):
            base = wid * EW + w * W
            pltpu.sync_copy(ei_hbm.at[pl.ds(base, W)], src_v)
            pltpu.sync_copy(ei_hbm.at[pl.ds(E + base, W)], dst_v)

            @pl.loop(0, W // L)
            def _vec(j):
                off = j * L
                s = src_v[pl.ds(off, L)]
                d = dst_v[pl.ds(off, L)]
                r = plsc.load_gather(nm_v, [s])
                c = plsc.load_gather(nm_v, [d])
                keep = (r >= 0) & (c >= 0)
                neg = jnp.full((L,), -1, jnp.int32)
                outr_v[pl.ds(off, L)] = jnp.where(keep, r, neg)
                outc_v[pl.ds(off, L)] = jnp.where(keep, c, neg)

            pltpu.sync_copy(outr_v, out_hbm.at[pl.ds(base, W)])
            pltpu.sync_copy(outc_v, out_hbm.at[pl.ds(E + base, W)])

    return k


def kernel(x, edge_index, weight):
    N = x.shape[0]
    E = edge_index.shape[1]
    K = (N * _K_FRAC_NUM + _K_FRAC_DEN - 1) // _K_FRAC_DEN
    score = jnp.tanh(
        jnp.sum(x * weight[None, :], axis=-1) / (jnp.linalg.norm(weight) + 1e-16)
    )
    _, perm = jax.lax.top_k(score, K)
    score_sel = jnp.take(score, perm, axis=0)
    x_out = jnp.take(x, perm, axis=0) * score_sel[:, None]
    node_map = (
        jnp.full((N,), -1, jnp.int32).at[perm].set(jnp.arange(K, dtype=jnp.int32))
    )
    new_edge_flat = _edge_remap(N, E)(node_map, edge_index.reshape(2 * E))
    return x_out, new_edge_flat.reshape(2, E), perm


# speedup vs baseline: 75.1966x; 1.1112x over previous
import functools

import jax
import jax.numpy as jnp
from jax import lax
from jax.experimental import pallas as pl
from jax.experimental.pallas import tpu as pltpu
from jax.experimental.pallas import tpu_sc as plsc

_NC, _NS, _L = 2, 16, 16
_NW = _NC * _NS
_RAD = 256


def _rank_topk(Np, K, interpret=False):
    T = _NS
    CH = Np // T
    LCH = CH // _L
    NV = CH // _L
    mesh = plsc.VectorSubcoreMesh(
        core_axis_name="c", subcore_axis_name="s", num_cores=_NC, num_subcores=_NS
    )
    i32 = jnp.int32

    @functools.partial(
        pl.kernel,
        out_type=(
            jax.ShapeDtypeStruct((Np,), i32),
            jax.ShapeDtypeStruct((Np,), i32),
            jax.ShapeDtypeStruct((Np,), i32),
        ),
        mesh=mesh,
        scratch_types=[
            pltpu.VMEM((CH,), jnp.float32),
            pltpu.VMEM((CH,), i32),
            pltpu.VMEM((CH,), i32),
            pltpu.VMEM((CH,), i32),
            pltpu.VMEM((CH,), i32),
            pltpu.VMEM((_RAD * _L,), i32),
            pltpu.VMEM((_RAD * _L,), i32),
            pltpu.VMEM((_RAD,), i32),
            pltpu.VMEM((_RAD,), i32),
            pltpu.VMEM((_RAD * T,), i32),
            pltpu.VMEM((_L,), i32),
            pltpu.VMEM_SHARED((Np,), i32),
            pltpu.VMEM_SHARED((Np,), i32),
            pltpu.VMEM_SHARED((Np,), i32),
            pltpu.VMEM_SHARED((Np,), i32),
            pltpu.VMEM_SHARED((Np,), i32),
            pltpu.VMEM_SHARED((Np,), i32),
            pltpu.VMEM_SHARED((Np,), i32),
            pltpu.VMEM_SHARED((_RAD * T,), i32),
        ],
        compiler_params=pltpu.CompilerParams(needs_layout_passes=False),
        interpret=interpret,
    )
    def k(score_hbm, nm_hbm, perm_hbm, ssc_hbm, sc_v, keys_v, vals_v, posb_v,
          nmb_v, hist_v, offs_v, base_v, ttot_v, g2all_v, tmp_v,
          sk0, sv0, sk1, sv1, nm_s, perm_s, ssc_s, g2_s):
        cid = lax.axis_index("c")
        t = lax.axis_index("s")
        iota = lax.iota(i32, _L)
        on0 = cid == 0

        @pl.when(on0)
        def _init():
            pltpu.sync_copy(score_hbm.at[pl.ds(t * CH, CH)], sc_v)

            @pl.loop(0, NV)
            def _i(j):
                off = j * _L
                bits = plsc.bitcast(sc_v[pl.ds(off, _L)], i32)
                d = jnp.where(bits < 0, bits, ~bits & 0x7FFFFFFF)
                keys_v[pl.ds(off, _L)] = d
                vals_v[pl.ds(off, _L)] = t * CH + off + iota

            pltpu.sync_copy(keys_v, sk0.at[pl.ds(t * CH, CH)])
            pltpu.sync_copy(vals_v, sv0.at[pl.ds(t * CH, CH)])

        plsc.subcore_barrier()

        bufs = [(sk0, sv0), (sk1, sv1)]
        for p in range(4):
            ink, inv = bufs[p % 2]
            outk, outv = bufs[1 - p % 2]
            shv = jnp.full((_L,), 8 * p, i32)

            @pl.when(on0)
            def _histphase(ink=ink, shv=shv):
                pltpu.sync_copy(ink.at[pl.ds(t * CH, CH)], keys_v)

                @pl.loop(0, _RAD)
                def _z(j):
                    hist_v[pl.ds(j * _L, _L)] = jnp.zeros((_L,), i32)

                @pl.loop(0, LCH)
                def _h(kk):
                    idx = iota * LCH + kk
                    key = plsc.load_gather(keys_v, [idx])
                    dig = lax.shift_right_logical(key, shv) & 0xFF
                    hidx = dig * _L + iota
                    old = plsc.load_gather(hist_v, [hidx])
                    plsc.store_scatter(hist_v, [hidx], old + 1)

                @pl.loop(0, _RAD // _L)
                def _r(g):
                    acc = jnp.zeros((_L,), i32)
                    for l in range(_L):
                        acc = acc + plsc.load_gather(
                            hist_v, [g * _RAD + iota * _L + l]
                        )
                    ttot_v[pl.ds(g * _L, _L)] = acc

                pltpu.sync_copy(ttot_v, g2_s.at[pl.ds(t * _RAD, _RAD)])

            plsc.subcore_barrier()

            @pl.when(on0)
            def _scanscatter(inv=inv, outk=outk, outv=outv, shv=shv, p=p):
                pltpu.sync_copy(g2_s, g2all_v)
                tvec = jnp.zeros((_L,), i32) + t

                @pl.loop(0, _RAD // _L, init_carry=jnp.zeros((_L,), i32))
                def _s(g, carry):
                    tot = jnp.zeros((_L,), i32)
                    tp = jnp.zeros((_L,), i32)
                    for tt in range(T):
                        row = g2all_v[pl.ds(tt * _RAD + g * _L, _L)]
                        tot = tot + row
                        m = jnp.full((_L,), tt, i32) < tvec
                        tp = tp + jnp.where(m, row, 0)
                    cs = plsc.cumsum(tot)
                    base_v[pl.ds(g * _L, _L)] = (cs - tot) + carry + tp
                    tmp_v[pl.ds(0, _L)] = cs
                    last = plsc.load_gather(tmp_v, [jnp.full((_L,), _L - 1, i32)])
                    return carry + last

                @pl.loop(0, _RAD)
                def _o(dg):
                    row = hist_v[pl.ds(dg * _L, _L)]
                    cs = plsc.cumsum(row)
                    b = plsc.load_gather(base_v, [jnp.zeros((_L,), i32) + dg])
                    offs_v[pl.ds(dg * _L, _L)] = (cs - row) + b

                pltpu.sync_copy(inv.at[pl.ds(t * CH, CH)], vals_v)

                @pl.loop(0, LCH)
                def _sc(kk):
                    idx = iota * LCH + kk
                    key = plsc.load_gather(keys_v, [idx])
                    dig = lax.shift_right_logical(key, shv) & 0xFF
                    hidx = dig * _L + iota
                    pos = plsc.load_gather(offs_v, [hidx])
                    plsc.store_scatter(offs_v, [hidx], pos + 1)
                    plsc.store_scatter(posb_v, [idx], pos)

                if p < 3:
                    pltpu.sync_copy(keys_v, outk.at[posb_v])
                    pltpu.sync_copy(vals_v, outv.at[posb_v])
                else:
                    @pl.loop(0, NV)
                    def _f(j):
                        off = j * _L
                        pos = posb_v[pl.ds(off, _L)]
                        nmb_v[pl.ds(off, _L)] = jnp.where(pos < K, pos, -1)

                    pltpu.sync_copy(nmb_v, nm_s.at[vals_v])
                    pltpu.sync_copy(vals_v, perm_s.at[posb_v])
                    pltpu.sync_copy(keys_v, ssc_s.at[posb_v])

            plsc.subcore_barrier()

        @pl.when(on0)
        def _out():
            sl = pl.ds(t * CH, CH)
            pltpu.sync_copy(nm_s.at[sl], nm_hbm.at[sl])
            pltpu.sync_copy(perm_s.at[sl], perm_hbm.at[sl])
            pltpu.sync_copy(ssc_s.at[sl], ssc_hbm.at[sl])

    return k


def _gather_edges(N, Np, K, KP, E, interpret=False):
    EW = E // _NW
    W = 2000 if EW % 2000 == 0 else EW
    NWIN = EW // W
    XW = 128
    NXW = KP // XW
    XIT = -(-NXW // _NW)
    i32 = jnp.int32
    f32 = jnp.float32
    mesh = plsc.VectorSubcoreMesh(
        core_axis_name="c", subcore_axis_name="s", num_cores=_NC, num_subcores=_NS
    )

    @functools.partial(
        pl.kernel,
        out_type=(
            jax.ShapeDtypeStruct((3 * KP,), f32),
            jax.ShapeDtypeStruct((2 * E,), i32),
        ),
        mesh=mesh,
        scratch_types=[
            pltpu.VMEM((Np,), i32),
            pltpu.VMEM((W,), i32),
            pltpu.VMEM((W,), i32),
            pltpu.VMEM((W,), i32),
            pltpu.VMEM((W,), i32),
            pltpu.VMEM((XW,), i32),
            pltpu.VMEM((XW,), i32),
            pltpu.VMEM((XW,), i32),
            pltpu.VMEM((3 * XW,), f32),
            pltpu.VMEM((3 * XW,), f32),
        ],
        compiler_params=pltpu.CompilerParams(needs_layout_passes=False),
        interpret=interpret,
    )
    def k(xflat_hbm, nm_hbm, perm_hbm, ssc_hbm, ei_hbm, xout_hbm, eio_hbm,
          nm_v, src_v, dst_v, outr_v, outc_v, idxb_v, ssk_v, ix3_v,
          rows3_v, outb_v):
        wid = lax.axis_index("s") * _NC + lax.axis_index("c")
        iota = lax.iota(i32, _L)

        @pl.loop(0, XIT)
        def _xw(i):
            w = wid + i * _NW

            @pl.when(w < NXW)
            def _do():
                base = w * XW
                pltpu.sync_copy(perm_hbm.at[pl.ds(base, XW)], idxb_v)
                pltpu.sync_copy(ssc_hbm.at[pl.ds(base, XW)], ssk_v)
                for c in range(3):
                    @pl.loop(0, XW // _L)
                    def _m(j, c=c):
                        off = j * _L
                        ix = idxb_v[pl.ds(off, _L)]
                        ix3_v[pl.ds(off, _L)] = ix * 3 + c

                    pltpu.sync_copy(
                        xflat_hbm.at[ix3_v],
                        rows3_v.at[pl.ds(c * XW, XW)],
                    )

                @pl.loop(0, 3 * XW // _L)
                def _mul(j):
                    jidx = iota + j * _L
                    sidx = jidx // 3
                    cidx = jidx - sidx * 3
                    d = plsc.load_gather(ssk_v, [sidx])
                    rb = jnp.where(d < 0, d, ~d & 0x7FFFFFFF)
                    f = plsc.bitcast(rb, f32)
                    v = plsc.load_gather(rows3_v, [cidx * XW + sidx])
                    outb_v[pl.ds(j * _L, _L)] = v * f

                pltpu.sync_copy(outb_v, xout_hbm.at[pl.ds(base * 3, 3 * XW)])

        pltpu.sync_copy(nm_hbm, nm_v)

        @pl.loop(0, NWIN)
        def _win(w):
            base = wid * EW + w * W
            pltpu.sync_copy(ei_hbm.at[pl.ds(base, W)], src_v)
            pltpu.sync_copy(ei_hbm.at[pl.ds(E + base, W)], dst_v)

            @pl.loop(0, W // _L)
            def _vec(j):
                off = j * _L
                s = src_v[pl.ds(off, _L)]
                d = dst_v[pl.ds(off, _L)]
                r = plsc.load_gather(nm_v, [s])
                c = plsc.load_gather(nm_v, [d])
                keep = (r >= 0) & (c >= 0)
                neg = jnp.full((_L,), -1, i32)
                outr_v[pl.ds(off, _L)] = jnp.where(keep, r, neg)
                outc_v[pl.ds(off, _L)] = jnp.where(keep, c, neg)

            pltpu.sync_copy(outr_v, eio_hbm.at[pl.ds(base, W)])
            pltpu.sync_copy(outc_v, eio_hbm.at[pl.ds(E + base, W)])

    return k


def kernel(x, edge_index, weight):
    N = x.shape[0]
    E = edge_index.shape[1]
    K = -(-N // 2)
    CHW = _NS * _L * 8
    Np = -(-N // CHW) * CHW
    XW = 128
    KP = -(-K // XW) * XW
    score = jnp.tanh(
        jnp.sum(x * weight[None, :], axis=-1) / (jnp.linalg.norm(weight) + 1e-16)
    )
    pad = jnp.full((Np - N,), jnp.asarray(-1, jnp.int32).view(jnp.float32))
    score_pad = jnp.concatenate([score, pad])
    nm_pad, perm_pad, ssc_pad = _rank_topk(Np, K)(score_pad)
    xout_flat, ei_out = _gather_edges(N, Np, K, KP, E)(
        x.reshape(3 * N), nm_pad, perm_pad, ssc_pad, edge_index.reshape(2 * E)
    )
    x_out = xout_flat[: 3 * K].reshape(K, 3)
    perm = perm_pad[:K]
    return x_out, ei_out.reshape(2, E), perm


# speedup vs baseline: 276.7556x; 3.6804x over previous
import functools

import jax
import jax.numpy as jnp
from jax import lax
from jax.experimental import pallas as pl
from jax.experimental.pallas import tpu as pltpu
from jax.experimental.pallas import tpu_sc as plsc

_NC, _NS, _L = 2, 16, 16
_NW = _NC * _NS
_RAD = 256


def _rank_topk(Np, K, interpret=False):
    T = _NS
    CH = Np // T
    LCH = CH // _L
    NV = CH // _L
    mesh = plsc.VectorSubcoreMesh(
        core_axis_name="c", subcore_axis_name="s", num_cores=_NC, num_subcores=_NS
    )
    i32 = jnp.int32

    @functools.partial(
        pl.kernel,
        out_type=(
            jax.ShapeDtypeStruct((Np,), i32),
            jax.ShapeDtypeStruct((Np,), i32),
            jax.ShapeDtypeStruct((Np,), i32),
        ),
        mesh=mesh,
        scratch_types=[
            pltpu.VMEM((CH,), jnp.float32),
            pltpu.VMEM((CH,), i32),
            pltpu.VMEM((CH,), i32),
            pltpu.VMEM((CH,), i32),
            pltpu.VMEM((CH,), i32),
            pltpu.VMEM((_RAD * _L,), i32),
            pltpu.VMEM((_RAD * _L,), i32),
            pltpu.VMEM((_RAD,), i32),
            pltpu.VMEM((_RAD,), i32),
            pltpu.VMEM((_RAD * T,), i32),
            pltpu.VMEM((_L,), i32),
            pltpu.VMEM_SHARED((Np,), i32),
            pltpu.VMEM_SHARED((Np,), i32),
            pltpu.VMEM_SHARED((Np,), i32),
            pltpu.VMEM_SHARED((Np,), i32),
            pltpu.VMEM_SHARED((Np,), i32),
            pltpu.VMEM_SHARED((Np,), i32),
            pltpu.VMEM_SHARED((Np,), i32),
            pltpu.VMEM_SHARED((_RAD * T,), i32),
        ],
        compiler_params=pltpu.CompilerParams(needs_layout_passes=False),
        interpret=interpret,
    )
    def k(score_hbm, nm_hbm, perm_hbm, ssc_hbm, sc_v, keys_v, vals_v, posb_v,
          nmb_v, hist_v, offs_v, base_v, ttot_v, g2all_v, tmp_v,
          sk0, sv0, sk1, sv1, nm_s, perm_s, ssc_s, g2_s):
        cid = lax.axis_index("c")
        t = lax.axis_index("s")
        iota = lax.iota(i32, _L)
        on0 = cid == 0

        @pl.when(on0)
        def _init():
            pltpu.sync_copy(score_hbm.at[pl.ds(t * CH, CH)], sc_v)

            @pl.loop(0, NV)
            def _i(j):
                off = j * _L
                bits = plsc.bitcast(sc_v[pl.ds(off, _L)], i32)
                d = jnp.where(bits < 0, bits, ~bits & 0x7FFFFFFF)
                keys_v[pl.ds(off, _L)] = d
                vals_v[pl.ds(off, _L)] = t * CH + off + iota

            pltpu.sync_copy(keys_v, sk0.at[pl.ds(t * CH, CH)])
            pltpu.sync_copy(vals_v, sv0.at[pl.ds(t * CH, CH)])

        plsc.subcore_barrier()

        bufs = [(sk0, sv0), (sk1, sv1)]
        for p in range(4):
            ink, inv = bufs[p % 2]
            outk, outv = bufs[1 - p % 2]
            shv = jnp.full((_L,), 8 * p, i32)

            @pl.when(on0)
            def _histphase(ink=ink, shv=shv):
                pltpu.sync_copy(ink.at[pl.ds(t * CH, CH)], keys_v)

                @pl.loop(0, _RAD)
                def _z(j):
                    hist_v[pl.ds(j * _L, _L)] = jnp.zeros((_L,), i32)

                @pl.loop(0, LCH)
                def _h(kk):
                    idx = iota * LCH + kk
                    key = plsc.load_gather(keys_v, [idx])
                    dig = lax.shift_right_logical(key, shv) & 0xFF
                    hidx = dig * _L + iota
                    old = plsc.load_gather(hist_v, [hidx])
                    plsc.store_scatter(hist_v, [hidx], old + 1)

                @pl.loop(0, _RAD // _L)
                def _r(g):
                    acc = jnp.zeros((_L,), i32)
                    for l in range(_L):
                        acc = acc + plsc.load_gather(
                            hist_v, [g * _RAD + iota * _L + l]
                        )
                    ttot_v[pl.ds(g * _L, _L)] = acc

                pltpu.sync_copy(ttot_v, g2_s.at[pl.ds(t * _RAD, _RAD)])

            plsc.subcore_barrier()

            @pl.when(on0)
            def _scanscatter(inv=inv, outk=outk, outv=outv, shv=shv, p=p):
                pltpu.sync_copy(g2_s, g2all_v)
                tvec = jnp.zeros((_L,), i32) + t

                @pl.loop(0, _RAD // _L, init_carry=jnp.zeros((_L,), i32))
                def _s(g, carry):
                    tot = jnp.zeros((_L,), i32)
                    tp = jnp.zeros((_L,), i32)
                    for tt in range(T):
                        row = g2all_v[pl.ds(tt * _RAD + g * _L, _L)]
                        tot = tot + row
                        m = jnp.full((_L,), tt, i32) < tvec
                        tp = tp + jnp.where(m, row, 0)
                    cs = plsc.cumsum(tot)
                    base_v[pl.ds(g * _L, _L)] = (cs - tot) + carry + tp
                    tmp_v[pl.ds(0, _L)] = cs
                    last = plsc.load_gather(tmp_v, [jnp.full((_L,), _L - 1, i32)])
                    return carry + last

                @pl.loop(0, _RAD)
                def _o(dg):
                    row = hist_v[pl.ds(dg * _L, _L)]
                    cs = plsc.cumsum(row)
                    b = plsc.load_gather(base_v, [jnp.zeros((_L,), i32) + dg])
                    offs_v[pl.ds(dg * _L, _L)] = (cs - row) + b

                pltpu.sync_copy(inv.at[pl.ds(t * CH, CH)], vals_v)

                @pl.loop(0, LCH)
                def _sc(kk):
                    idx = iota * LCH + kk
                    key = plsc.load_gather(keys_v, [idx])
                    dig = lax.shift_right_logical(key, shv) & 0xFF
                    hidx = dig * _L + iota
                    pos = plsc.load_gather(offs_v, [hidx])
                    plsc.store_scatter(offs_v, [hidx], pos + 1)
                    plsc.store_scatter(posb_v, [idx], pos)

                if p < 3:
                    pltpu.sync_copy(keys_v, outk.at[posb_v])
                    pltpu.sync_copy(vals_v, outv.at[posb_v])
                else:
                    @pl.loop(0, NV)
                    def _f(j):
                        off = j * _L
                        pos = posb_v[pl.ds(off, _L)]
                        nmb_v[pl.ds(off, _L)] = jnp.where(pos < K, pos, -1)

                    pltpu.sync_copy(nmb_v, nm_s.at[vals_v])
                    pltpu.sync_copy(vals_v, perm_s.at[posb_v])
                    pltpu.sync_copy(keys_v, ssc_s.at[posb_v])

            plsc.subcore_barrier()

        @pl.when(on0)
        def _out():
            sl = pl.ds(t * CH, CH)
            pltpu.sync_copy(nm_s.at[sl], nm_hbm.at[sl])
            pltpu.sync_copy(perm_s.at[sl], perm_hbm.at[sl])
            pltpu.sync_copy(ssc_s.at[sl], ssc_hbm.at[sl])

    return k


def _gather_edges(N, Np, K, KP, E, interpret=False):
    EW = E // _NW
    W = 2000 if EW % 2000 == 0 else EW
    NWIN = EW // W
    XW = 128
    NXW = KP // XW
    XIT = -(-NXW // _NW)
    i32 = jnp.int32
    f32 = jnp.float32
    mesh = plsc.VectorSubcoreMesh(
        core_axis_name="c", subcore_axis_name="s", num_cores=_NC, num_subcores=_NS
    )

    @functools.partial(
        pl.kernel,
        out_type=(
            jax.ShapeDtypeStruct((3 * KP,), f32),
            jax.ShapeDtypeStruct((2 * E,), i32),
        ),
        mesh=mesh,
        scratch_types=[
            pltpu.VMEM((Np,), i32),
            pltpu.VMEM((W,), i32),
            pltpu.VMEM((W,), i32),
            pltpu.VMEM((W,), i32),
            pltpu.VMEM((W,), i32),
            pltpu.VMEM((W,), i32),
            pltpu.VMEM((W,), i32),
            pltpu.VMEM((W,), i32),
            pltpu.VMEM((W,), i32),
            pltpu.VMEM((XW,), i32),
            pltpu.VMEM((XW,), i32),
            pltpu.VMEM((XW,), i32),
            pltpu.VMEM((3 * XW,), f32),
            pltpu.VMEM((3 * XW,), f32),
            pltpu.SemaphoreType.DMA,
            pltpu.SemaphoreType.DMA,
            pltpu.SemaphoreType.DMA,
            pltpu.SemaphoreType.DMA,
        ],
        compiler_params=pltpu.CompilerParams(needs_layout_passes=False),
        interpret=interpret,
    )
    def k(xflat_hbm, nm_hbm, perm_hbm, ssc_hbm, ei_hbm, xout_hbm, eio_hbm,
          nm_v, srcA, dstA, outrA, outcA, srcB, dstB, outrB, outcB,
          idxb_v, ssk_v, ix3_v, rows3_v, outb_v,
          inA_sem, inB_sem, outA_sem, outB_sem):
        wid = lax.axis_index("s") * _NC + lax.axis_index("c")
        iota = lax.iota(i32, _L)

        @pl.loop(0, XIT)
        def _xw(i):
            w = wid + i * _NW

            @pl.when(w < NXW)
            def _do():
                base = w * XW
                pltpu.sync_copy(perm_hbm.at[pl.ds(base, XW)], idxb_v)
                pltpu.sync_copy(ssc_hbm.at[pl.ds(base, XW)], ssk_v)
                for c in range(3):
                    @pl.loop(0, XW // _L)
                    def _m(j, c=c):
                        off = j * _L
                        ix = idxb_v[pl.ds(off, _L)]
                        ix3_v[pl.ds(off, _L)] = ix * 3 + c

                    pltpu.sync_copy(
                        xflat_hbm.at[ix3_v],
                        rows3_v.at[pl.ds(c * XW, XW)],
                    )

                @pl.loop(0, 3 * XW // _L)
                def _mul(j):
                    jidx = iota + j * _L
                    sidx = jidx // 3
                    cidx = jidx - sidx * 3
                    d = plsc.load_gather(ssk_v, [sidx])
                    rb = jnp.where(d < 0, d, ~d & 0x7FFFFFFF)
                    f = plsc.bitcast(rb, f32)
                    v = plsc.load_gather(rows3_v, [cidx * XW + sidx])
                    outb_v[pl.ds(j * _L, _L)] = v * f

                pltpu.sync_copy(outb_v, xout_hbm.at[pl.ds(base * 3, 3 * XW)])

        pltpu.sync_copy(nm_hbm, nm_v)
        rings = [
            (srcA, dstA, outrA, outcA, inA_sem, outA_sem),
            (srcB, dstB, outrB, outcB, inB_sem, outB_sem),
        ]

        def in_slices(w):
            base = wid * EW + w * W
            return ei_hbm.at[pl.ds(base, W)], ei_hbm.at[pl.ds(E + base, W)]

        def out_slices(w):
            base = wid * EW + w * W
            return eio_hbm.at[pl.ds(base, W)], eio_hbm.at[pl.ds(E + base, W)]

        for par in (0, 1):
            sv, dv, _, _, isem, _ = rings[par]
            s_sl, d_sl = in_slices(jnp.int32(par))
            pltpu.async_copy(s_sl, sv, isem)
            pltpu.async_copy(d_sl, dv, isem)

        @pl.loop(0, NWIN // 2)
        def _win2(h):
            for par in (0, 1):
                sv, dv, orv, ocv, isem, osem = rings[par]
                w = h * 2 + par
                s_sl, d_sl = in_slices(w)
                pltpu.make_async_copy(s_sl, sv, isem).wait()
                pltpu.make_async_copy(d_sl, dv, isem).wait()

                @pl.when(h > 0)
                def _drain(par=par, orv=orv, ocv=ocv, osem=osem, w=w):
                    po_sl = out_slices(w - 2)
                    pltpu.make_async_copy(orv, po_sl[0], osem).wait()
                    pltpu.make_async_copy(ocv, po_sl[1], osem).wait()

                @pl.loop(0, W // _L)
                def _vec(j, sv=sv, dv=dv, orv=orv, ocv=ocv):
                    off = j * _L
                    s = sv[pl.ds(off, _L)]
                    d = dv[pl.ds(off, _L)]
                    r = plsc.load_gather(nm_v, [s])
                    c = plsc.load_gather(nm_v, [d])
                    keep = (r >= 0) & (c >= 0)
                    neg = jnp.full((_L,), -1, i32)
                    orv[pl.ds(off, _L)] = jnp.where(keep, r, neg)
                    ocv[pl.ds(off, _L)] = jnp.where(keep, c, neg)

                o_sl = out_slices(w)
                pltpu.async_copy(orv, o_sl[0], osem)
                pltpu.async_copy(ocv, o_sl[1], osem)

                @pl.when(w + 2 < NWIN)
                def _pref(par=par, sv=sv, dv=dv, isem=isem, w=w):
                    n_sl = in_slices(w + 2)
                    pltpu.async_copy(n_sl[0], sv, isem)
                    pltpu.async_copy(n_sl[1], dv, isem)

        for par in (0, 1):
            _, _, orv, ocv, _, osem = rings[par]
            lo_sl = out_slices(jnp.int32(NWIN - 2 + par))
            pltpu.make_async_copy(orv, lo_sl[0], osem).wait()
            pltpu.make_async_copy(ocv, lo_sl[1], osem).wait()

    return k


def kernel(x, edge_index, weight):
    N = x.shape[0]
    E = edge_index.shape[1]
    K = -(-N // 2)
    CHW = _NS * _L * 8
    Np = -(-N // CHW) * CHW
    XW = 128
    KP = -(-K // XW) * XW
    score = jnp.tanh(
        jnp.sum(x * weight[None, :], axis=-1) / (jnp.linalg.norm(weight) + 1e-16)
    )
    pad = jnp.full((Np - N,), jnp.asarray(-1, jnp.int32).view(jnp.float32))
    score_pad = jnp.concatenate([score, pad])
    nm_pad, perm_pad, ssc_pad = _rank_topk(Np, K)(score_pad)
    xout_flat, ei_out = _gather_edges(N, Np, K, KP, E)(
        x.reshape(3 * N), nm_pad, perm_pad, ssc_pad, edge_index.reshape(2 * E)
    )
    x_out = xout_flat[: 3 * K].reshape(K, 3)
    perm = perm_pad[:K]
    new_edge_index = jnp.concatenate([ei_out[None, :E], ei_out[None, E:]], axis=0)
    return x_out, new_edge_index, perm


# speedup vs baseline: 459.5227x; 1.6604x over previous
import functools

import jax
import jax.numpy as jnp
from jax import lax
from jax.experimental import pallas as pl
from jax.experimental.pallas import tpu as pltpu
from jax.experimental.pallas import tpu_sc as plsc

_NC, _NS, _L = 2, 16, 16
_NW = _NC * _NS
_RAD = 256


def _rank_topk(Np, K, interpret=False):
    T = _NS
    CH = Np // T
    LCH = CH // _L
    NV = CH // _L
    mesh = plsc.VectorSubcoreMesh(
        core_axis_name="c", subcore_axis_name="s", num_cores=_NC, num_subcores=_NS
    )
    i32 = jnp.int32

    @functools.partial(
        pl.kernel,
        out_type=(
            jax.ShapeDtypeStruct((Np,), i32),
            jax.ShapeDtypeStruct((Np,), i32),
            jax.ShapeDtypeStruct((Np,), i32),
        ),
        mesh=mesh,
        scratch_types=[
            pltpu.VMEM((CH,), jnp.float32),
            pltpu.VMEM((CH,), i32),
            pltpu.VMEM((CH,), i32),
            pltpu.VMEM((CH,), i32),
            pltpu.VMEM((CH,), i32),
            pltpu.VMEM((_RAD * _L,), i32),
            pltpu.VMEM((_RAD * _L,), i32),
            pltpu.VMEM((_RAD,), i32),
            pltpu.VMEM((_RAD,), i32),
            pltpu.VMEM((_RAD * T,), i32),
            pltpu.VMEM((_L,), i32),
            pltpu.VMEM_SHARED((Np,), i32),
            pltpu.VMEM_SHARED((Np,), i32),
            pltpu.VMEM_SHARED((Np,), i32),
            pltpu.VMEM_SHARED((Np,), i32),
            pltpu.VMEM_SHARED((Np,), i32),
            pltpu.VMEM_SHARED((Np,), i32),
            pltpu.VMEM_SHARED((Np,), i32),
            pltpu.VMEM_SHARED((_RAD * T,), i32),
        ],
        compiler_params=pltpu.CompilerParams(needs_layout_passes=False),
        interpret=interpret,
    )
    def k(score_hbm, nm_hbm, perm_hbm, ssc_hbm, sc_v, keys_v, vals_v, posb_v,
          nmb_v, hist_v, offs_v, base_v, ttot_v, g2all_v, tmp_v,
          sk0, sv0, sk1, sv1, nm_s, perm_s, ssc_s, g2_s):
        cid = lax.axis_index("c")
        t = lax.axis_index("s")
        iota = lax.iota(i32, _L)
        on0 = cid == 0

        @pl.when(on0)
        def _init():
            pltpu.sync_copy(score_hbm.at[pl.ds(t * CH, CH)], sc_v)

            @pl.loop(0, NV)
            def _i(j):
                off = j * _L
                bits = plsc.bitcast(sc_v[pl.ds(off, _L)], i32)
                d = jnp.where(bits < 0, bits, ~bits & 0x7FFFFFFF)
                keys_v[pl.ds(off, _L)] = d
                vals_v[pl.ds(off, _L)] = t * CH + off + iota

            pltpu.sync_copy(keys_v, sk0.at[pl.ds(t * CH, CH)])
            pltpu.sync_copy(vals_v, sv0.at[pl.ds(t * CH, CH)])

        plsc.subcore_barrier()

        bufs = [(sk0, sv0), (sk1, sv1)]
        for p in range(4):
            ink, inv = bufs[p % 2]
            outk, outv = bufs[1 - p % 2]
            shv = jnp.full((_L,), 8 * p, i32)

            @pl.when(on0)
            def _histphase(ink=ink, shv=shv):
                pltpu.sync_copy(ink.at[pl.ds(t * CH, CH)], keys_v)

                @pl.loop(0, _RAD)
                def _z(j):
                    hist_v[pl.ds(j * _L, _L)] = jnp.zeros((_L,), i32)

                @pl.loop(0, LCH)
                def _h(kk):
                    idx = iota * LCH + kk
                    key = plsc.load_gather(keys_v, [idx])
                    dig = lax.shift_right_logical(key, shv) & 0xFF
                    hidx = dig * _L + iota
                    old = plsc.load_gather(hist_v, [hidx])
                    plsc.store_scatter(hist_v, [hidx], old + 1)

                @pl.loop(0, _RAD // _L)
                def _r(g):
                    acc = jnp.zeros((_L,), i32)
                    for l in range(_L):
                        acc = acc + plsc.load_gather(
                            hist_v, [g * _RAD + iota * _L + l]
                        )
                    ttot_v[pl.ds(g * _L, _L)] = acc

                pltpu.sync_copy(ttot_v, g2_s.at[pl.ds(t * _RAD, _RAD)])

            plsc.subcore_barrier()

            @pl.when(on0)
            def _scanscatter(inv=inv, outk=outk, outv=outv, shv=shv, p=p):
                pltpu.sync_copy(g2_s, g2all_v)
                tvec = jnp.zeros((_L,), i32) + t

                @pl.loop(0, _RAD // _L, init_carry=jnp.zeros((_L,), i32))
                def _s(g, carry):
                    tot = jnp.zeros((_L,), i32)
                    tp = jnp.zeros((_L,), i32)
                    for tt in range(T):
                        row = g2all_v[pl.ds(tt * _RAD + g * _L, _L)]
                        tot = tot + row
                        m = jnp.full((_L,), tt, i32) < tvec
                        tp = tp + jnp.where(m, row, 0)
                    cs = plsc.cumsum(tot)
                    base_v[pl.ds(g * _L, _L)] = (cs - tot) + carry + tp
                    tmp_v[pl.ds(0, _L)] = cs
                    last = plsc.load_gather(tmp_v, [jnp.full((_L,), _L - 1, i32)])
                    return carry + last

                @pl.loop(0, _RAD)
                def _o(dg):
                    row = hist_v[pl.ds(dg * _L, _L)]
                    cs = plsc.cumsum(row)
                    b = plsc.load_gather(base_v, [jnp.zeros((_L,), i32) + dg])
                    offs_v[pl.ds(dg * _L, _L)] = (cs - row) + b

                pltpu.sync_copy(inv.at[pl.ds(t * CH, CH)], vals_v)

                @pl.loop(0, LCH)
                def _sc(kk):
                    idx = iota * LCH + kk
                    key = plsc.load_gather(keys_v, [idx])
                    dig = lax.shift_right_logical(key, shv) & 0xFF
                    hidx = dig * _L + iota
                    pos = plsc.load_gather(offs_v, [hidx])
                    plsc.store_scatter(offs_v, [hidx], pos + 1)
                    plsc.store_scatter(posb_v, [idx], pos)

                if p < 3:
                    pltpu.sync_copy(keys_v, outk.at[posb_v])
                    pltpu.sync_copy(vals_v, outv.at[posb_v])
                else:
                    @pl.loop(0, NV)
                    def _f(j):
                        off = j * _L
                        pos = posb_v[pl.ds(off, _L)]
                        nmb_v[pl.ds(off, _L)] = jnp.where(pos < K, pos, -1)

                    pltpu.sync_copy(nmb_v, nm_s.at[vals_v])
                    pltpu.sync_copy(vals_v, perm_s.at[posb_v])
                    pltpu.sync_copy(keys_v, ssc_s.at[posb_v])

            plsc.subcore_barrier()

        @pl.when(on0)
        def _out():
            sl = pl.ds(t * CH, CH)
            pltpu.sync_copy(nm_s.at[sl], nm_hbm.at[sl])
            pltpu.sync_copy(perm_s.at[sl], perm_hbm.at[sl])
            pltpu.sync_copy(ssc_s.at[sl], ssc_hbm.at[sl])

    return k


def _gather_edges(N, Np, K, KP, E, interpret=False):
    W = 2048
    NWIN = E // W
    NIT = -(-NWIN // _NW)
    XW = 128
    NXW = KP // XW
    XIT = -(-NXW // _NW)
    i32 = jnp.int32
    f32 = jnp.float32
    mesh = plsc.VectorSubcoreMesh(
        core_axis_name="c", subcore_axis_name="s", num_cores=_NC, num_subcores=_NS
    )

    @functools.partial(
        pl.kernel,
        out_type=(
            jax.ShapeDtypeStruct((3 * KP,), f32),
            jax.ShapeDtypeStruct((2, E), i32),
        ),
        mesh=mesh,
        scratch_types=[
            pltpu.VMEM((Np,), i32),
            pltpu.VMEM((2, W), i32),
            pltpu.VMEM((2, W), i32),
            pltpu.VMEM((2, W), i32),
            pltpu.VMEM((2, W), i32),
            pltpu.VMEM((XW,), i32),
            pltpu.VMEM((XW,), i32),
            pltpu.VMEM((XW,), i32),
            pltpu.VMEM((3 * XW,), f32),
            pltpu.VMEM((3 * XW,), f32),
            pltpu.SemaphoreType.DMA,
            pltpu.SemaphoreType.DMA,
            pltpu.SemaphoreType.DMA,
            pltpu.SemaphoreType.DMA,
        ],
        compiler_params=pltpu.CompilerParams(needs_layout_passes=False),
        interpret=interpret,
    )
    def k(xflat_hbm, nm_hbm, perm_hbm, ssc_hbm, ei_hbm, xout_hbm, eio_hbm,
          nm_v, inA, outA, inB, outB,
          idxb_v, ssk_v, ix3_v, rows3_v, outb_v,
          inA_sem, inB_sem, outA_sem, outB_sem):
        wid = lax.axis_index("s") * _NC + lax.axis_index("c")
        iota = lax.iota(i32, _L)

        @pl.loop(0, XIT)
        def _xw(i):
            w = wid + i * _NW

            @pl.when(w < NXW)
            def _do():
                base = w * XW
                pltpu.sync_copy(perm_hbm.at[pl.ds(base, XW)], idxb_v)
                pltpu.sync_copy(ssc_hbm.at[pl.ds(base, XW)], ssk_v)
                for c in range(3):
                    @pl.loop(0, XW // _L)
                    def _m(j, c=c):
                        off = j * _L
                        ix = idxb_v[pl.ds(off, _L)]
                        ix3_v[pl.ds(off, _L)] = ix * 3 + c

                    pltpu.sync_copy(
                        xflat_hbm.at[ix3_v],
                        rows3_v.at[pl.ds(c * XW, XW)],
                    )

                @pl.loop(0, 3 * XW // _L)
                def _mul(j):
                    jidx = iota + j * _L
                    sidx = jidx // 3
                    cidx = jidx - sidx * 3
                    d = plsc.load_gather(ssk_v, [sidx])
                    rb = jnp.where(d < 0, d, ~d & 0x7FFFFFFF)
                    f = plsc.bitcast(rb, f32)
                    v = plsc.load_gather(rows3_v, [cidx * XW + sidx])
                    outb_v[pl.ds(j * _L, _L)] = v * f

                pltpu.sync_copy(outb_v, xout_hbm.at[pl.ds(base * 3, 3 * XW)])

        pltpu.sync_copy(nm_hbm, nm_v)
        rings = [(inA, outA, inA_sem, outA_sem), (inB, outB, inB_sem, outB_sem)]
        RING = NIT - 2

        def in_sl(w):
            return ei_hbm.at[:, pl.ds(w * W, W)]

        def out_sl(w):
            return eio_hbm.at[:, pl.ds(w * W, W)]

        def compute(inb, outb):
            @pl.loop(0, W // _L)
            def _vec(j):
                off = j * _L
                s = inb[0, pl.ds(off, _L)]
                d = inb[1, pl.ds(off, _L)]
                r = plsc.load_gather(nm_v, [s])
                c = plsc.load_gather(nm_v, [d])
                keep = (r >= 0) & (c >= 0)
                neg = jnp.full((_L,), -1, i32)
                outb[0, pl.ds(off, _L)] = jnp.where(keep, r, neg)
                outb[1, pl.ds(off, _L)] = jnp.where(keep, c, neg)

        for par in (0, 1):
            inb, _, isem, _ = rings[par]
            pltpu.async_copy(in_sl(wid + par * _NW), inb, isem)

        @pl.loop(0, RING // 2)
        def _win2(h):
            for par in (0, 1):
                inb, outb, isem, osem = rings[par]
                i = h * 2 + par
                w = wid + i * _NW
                pltpu.make_async_copy(in_sl(w), inb, isem).wait()

                @pl.when(h > 0)
                def _drain(outb=outb, osem=osem, w=w):
                    pltpu.make_async_copy(outb, out_sl(w - 2 * _NW), osem).wait()

                compute(inb, outb)
                pltpu.async_copy(outb, out_sl(w), osem)

                @pl.when(w + 2 * _NW < NWIN * 1)
                def _pref(inb=inb, isem=isem, w=w):
                    pltpu.async_copy(in_sl(w + 2 * _NW), inb, isem)

        wt0 = wid + (NIT - 2) * _NW
        wt1 = wid + (NIT - 1) * _NW
        v0 = wt0 < NWIN
        v1 = wt1 < NWIN

        @pl.when(v0)
        def _t0a():
            pltpu.make_async_copy(in_sl(wt0), inA, inA_sem).wait()
        pltpu.make_async_copy(outA, out_sl(wt0 - 2 * _NW), outA_sem).wait()

        @pl.when(v0)
        def _t0b():
            compute(inA, outA)
            pltpu.async_copy(outA, out_sl(wt0), outA_sem)

        @pl.when(v1)
        def _t1a():
            pltpu.make_async_copy(in_sl(wt1), inB, inB_sem).wait()
        pltpu.make_async_copy(outB, out_sl(wt1 - 2 * _NW), outB_sem).wait()

        @pl.when(v1)
        def _t1b():
            compute(inB, outB)
            pltpu.async_copy(outB, out_sl(wt1), outB_sem)

        @pl.when(v0)
        def _t0c():
            pltpu.make_async_copy(outA, out_sl(wt0), outA_sem).wait()

        @pl.when(v1)
        def _t1c():
            pltpu.make_async_copy(outB, out_sl(wt1), outB_sem).wait()

    return k


def kernel(x, edge_index, weight):
    N = x.shape[0]
    E = edge_index.shape[1]
    K = -(-N // 2)
    CHW = _NS * _L * 8
    Np = -(-N // CHW) * CHW
    XW = 128
    KP = -(-K // XW) * XW
    score = jnp.tanh(
        jnp.sum(x * weight[None, :], axis=-1) / (jnp.linalg.norm(weight) + 1e-16)
    )
    pad = jnp.full((Np - N,), jnp.asarray(-1, jnp.int32).view(jnp.float32))
    score_pad = jnp.concatenate([score, pad])
    nm_pad, perm_pad, ssc_pad = _rank_topk(Np, K)(score_pad)
    xout_flat, new_edge_index = _gather_edges(N, Np, K, KP, E)(
        x.reshape(3 * N), nm_pad, perm_pad, ssc_pad, edge_index
    )
    x_out = xout_flat[: 3 * K].reshape(K, 3)
    perm = perm_pad[:K]
    return x_out, new_edge_index, perm
